# Initial kernel scaffold; baseline (speedup 1.0000x reference)
import jax
import jax.numpy as jnp
from jax.experimental import pallas as pl


def kernel(pos, h, edge_index, gamma, beta, rm, rv, W1, b1, W2, b2, Wq0, Wk0, Wv0, Wo0, Wm1a, bm1a, Wm2a, bm2a, Wqa, Wka, Wva, Woa, Wm1b, bm1b, Wm2b, bm2b, Wqb, Wkb, Wvb, Wob):
    raise NotImplementedError("write your pallas kernel here")


# speedup vs baseline: 2.1263x; 2.1263x over previous
import functools
import math

import jax
import jax.numpy as jnp
from jax.experimental import pallas as pl

_N = 10000
_K = 7


def _stem_kernel(h_ref, gamma_ref, beta_ref, rm_ref, rv_ref, W1_ref, b1_ref,
                 W2_ref, b2_ref, Wq_ref, Wk_ref, Wv_ref,
                 x_ref, q_ref, k_ref, v_ref):
    h = h_ref[...]
    x = (h - rm_ref[...]) / jnp.sqrt(rv_ref[...] + 1e-5) * gamma_ref[...] + beta_ref[...]
    x = jnp.maximum(x @ W1_ref[...] + b1_ref[...], 0.0) @ W2_ref[...] + b2_ref[...]
    x_ref[...] = x
    q_ref[...] = x @ Wq_ref[...]
    k_ref[...] = x @ Wk_ref[...]
    v_ref[...] = x @ Wv_ref[...]


def _gtout_kernel(x_ref, agg_ref, Wo_ref, o_ref):
    o_ref[...] = x_ref[...] + agg_ref[...] @ Wo_ref[...]


def _conv_kernel(up_ref, dp_ref, dh_ref, Wm1_ref, bm1_ref, Wm2_ref, bm2_ref,
                 Wq_ref, Wk_ref, Wv_ref,
                 newh_ref, q_ref, k_ref, v_ref, *, n_down, kk):
    up = up_ref[...]
    dp = dp_ref[...]
    dh = dh_ref[...]
    upn = jnp.sum(up * up, axis=1, keepdims=True)
    dpn = jnp.sum(dp * dp, axis=1, keepdims=True)
    d2 = upn + dpn.T - 2.0 * (up @ dp.T)
    cols = jax.lax.broadcasted_iota(jnp.int32, d2.shape, 1)
    Wm1 = Wm1_ref[...]
    Wm1_rel = Wm1[:3, :]
    Wm1_h = Wm1[3:, :]
    acc = jnp.full((up.shape[0], Wm2_ref.shape[1]), -jnp.inf, dtype=jnp.float32)
    for _ in range(kk):
        minv = jnp.min(d2, axis=1, keepdims=True)
        cand = jnp.where(d2 == minv, cols, n_down)
        sel = jnp.min(cand, axis=1, keepdims=True)
        onehot = (cols == sel).astype(jnp.float32)
        rel = onehot @ dp - up
        nh = onehot @ dh
        m = jnp.maximum(rel @ Wm1_rel + nh @ Wm1_h + bm1_ref[...], 0.0)
        m = jnp.maximum(m @ Wm2_ref[...] + bm2_ref[...], 0.0)
        acc = jnp.maximum(acc, m)
        d2 = jnp.where(cols == sel, jnp.inf, d2)
    newh_ref[...] = acc
    q_ref[...] = acc @ Wq_ref[...]
    k_ref[...] = acc @ Wk_ref[...]
    v_ref[...] = acc @ Wv_ref[...]


def _attn_kernel(up_ref, ap_ref, nh_ref, q_ref, kf_ref, vf_ref, Wo_ref,
                 o_ref, *, n_up, kk, bq, hid):
    up = up_ref[...]
    ap = ap_ref[...]
    upn = jnp.sum(up * up, axis=1, keepdims=True)
    apn = jnp.sum(ap * ap, axis=1, keepdims=True)
    d2 = upn + apn.T - 2.0 * (up @ ap.T)
    cols = jax.lax.broadcasted_iota(jnp.int32, d2.shape, 1)
    rows = jax.lax.broadcasted_iota(jnp.int32, d2.shape, 0) + pl.program_id(0) * bq
    d2 = jnp.where(cols == rows, d2 + 1e10, d2)
    qb = q_ref[...]
    kf = kf_ref[...]
    vf = vf_ref[...]
    scs = []
    nvs = []
    for _ in range(kk):
        minv = jnp.min(d2, axis=1, keepdims=True)
        cand = jnp.where(d2 == minv, cols, n_up)
        sel = jnp.min(cand, axis=1, keepdims=True)
        onehot = (cols == sel).astype(jnp.float32)
        nk = onehot @ kf
        nvs.append(onehot @ vf)
        scs.append(jnp.sum(qb * nk, axis=1, keepdims=True) / jnp.sqrt(float(hid)))
        d2 = jnp.where(cols == sel, jnp.inf, d2)
    sc = jnp.concatenate(scs, axis=1)
    mx = jnp.max(sc, axis=1, keepdims=True)
    e = jnp.exp(sc - mx)
    den = jnp.sum(e, axis=1, keepdims=True) + 1e-9
    agg = nvs[0] * e[:, 0:1]
    for j in range(1, kk):
        agg = agg + nvs[j] * e[:, j:j + 1]
    agg = agg / den
    o_ref[...] = nh_ref[...] + agg @ Wo_ref[...]


def _full(shape):
    return pl.BlockSpec(shape, lambda i: (0,) * len(shape))


def _rowblk(bq, w):
    return pl.BlockSpec((bq, w), lambda i: (i, 0))


def _level(up_pos, down_pos, down_h, Wm1, bm1, Wm2, bm2, Wq, Wk, Wv, Wo, bq):
    n_up = up_pos.shape[0]
    n_down = down_pos.shape[0]
    hin = down_h.shape[1]
    hid = Wm2.shape[1]
    grid = (pl.cdiv(n_up, bq),)
    conv = pl.pallas_call(
        functools.partial(_conv_kernel, n_down=n_down, kk=_K),
        grid=grid,
        in_specs=[
            _rowblk(bq, 3), _full((n_down, 3)), _full((n_down, hin)),
            _full(Wm1.shape), _full((1, hid)), _full(Wm2.shape), _full((1, hid)),
            _full(Wq.shape), _full(Wk.shape), _full(Wv.shape),
        ],
        out_specs=[_rowblk(bq, hid)] * 4,
        out_shape=[jax.ShapeDtypeStruct((n_up, hid), jnp.float32)] * 4,
    )
    newh, q, kmat, v = conv(up_pos, down_pos, down_h, Wm1, bm1.reshape(1, -1),
                            Wm2, bm2.reshape(1, -1), Wq, Wk, Wv)
    attn = pl.pallas_call(
        functools.partial(_attn_kernel, n_up=n_up, kk=_K, bq=bq, hid=hid),
        grid=grid,
        in_specs=[
            _rowblk(bq, 3), _full((n_up, 3)), _rowblk(bq, hid), _rowblk(bq, hid),
            _full((n_up, hid)), _full((n_up, hid)), _full(Wo.shape),
        ],
        out_specs=_rowblk(bq, hid),
        out_shape=jax.ShapeDtypeStruct((n_up, hid), jnp.float32),
    )
    return attn(up_pos, up_pos, newh, q, kmat, v, Wo)


def kernel(pos, h, edge_index, gamma, beta, rm, rv, W1, b1, W2, b2,
           Wq0, Wk0, Wv0, Wo0, Wm1a, bm1a, Wm2a, bm2a, Wqa, Wka, Wva, Woa,
           Wm1b, bm1b, Wm2b, bm2b, Wqb, Wkb, Wvb, Wob):
    n = pos.shape[0]
    hid0 = W2.shape[1]
    bq0 = 512
    stem = pl.pallas_call(
        _stem_kernel,
        grid=(pl.cdiv(n, bq0),),
        in_specs=[
            _rowblk(bq0, h.shape[1]),
            _full((1, h.shape[1])), _full((1, h.shape[1])),
            _full((1, h.shape[1])), _full((1, h.shape[1])),
            _full(W1.shape), _full((1, W1.shape[1])),
            _full(W2.shape), _full((1, W2.shape[1])),
            _full(Wq0.shape), _full(Wk0.shape), _full(Wv0.shape),
        ],
        out_specs=[_rowblk(bq0, hid0)] * 4,
        out_shape=[jax.ShapeDtypeStruct((n, hid0), jnp.float32)] * 4,
    )
    x, q0, k0, v0 = stem(h, gamma.reshape(1, -1), beta.reshape(1, -1),
                         rm.reshape(1, -1), rv.reshape(1, -1),
                         W1, b1.reshape(1, -1), W2, b2.reshape(1, -1),
                         Wq0, Wk0, Wv0)

    src = edge_index[0].astype(jnp.int32)
    dst = edge_index[1].astype(jnp.int32)
    sc = jnp.sum(q0[dst] * k0[src], axis=-1) / jnp.sqrt(float(hid0))
    m = jax.ops.segment_max(sc, dst, num_segments=n)
    m = jnp.where(jnp.isfinite(m), m, 0.0)
    e = jnp.exp(sc - m[dst])
    den = jax.ops.segment_sum(e, dst, num_segments=n)
    agg = jax.ops.segment_sum(e[:, None] * v0[src], dst, num_segments=n)
    agg = agg / (den[:, None] + 1e-9)

    gtout = pl.pallas_call(
        _gtout_kernel,
        grid=(pl.cdiv(n, bq0),),
        in_specs=[_rowblk(bq0, hid0), _rowblk(bq0, hid0), _full(Wo0.shape)],
        out_specs=_rowblk(bq0, hid0),
        out_shape=jax.ShapeDtypeStruct((n, hid0), jnp.float32),
    )
    h1 = gtout(x, agg, Wo0)

    n1 = int(math.floor(n * 0.25))
    h2 = _level(pos[:n1], pos[n1:], h1[n1:],
                Wm1a, bm1a, Wm2a, bm2a, Wqa, Wka, Wva, Woa, bq=256)
    pos1 = pos[:n1]
    n2 = int(math.floor(n1 * 0.25))
    h3 = _level(pos1[:n2], pos1[n2:], h2[n2:],
                Wm1b, bm1b, Wm2b, bm2b, Wqb, Wkb, Wvb, Wob, bq=128)
    return h3
